# baseline (speedup 1.0000x reference)
import jax
import jax.numpy as jnp
from jax import lax
from jax.experimental import pallas as pl
from jax.experimental.pallas import tpu as pltpu
from jax.experimental.pallas import tpu_sc as plsc

_N = 100000
_NV = _N // 16
_NC = 2
_NW = 32
_BASE = _NV // _NW
_REM = _NV - _BASE * _NW
_MAXV = _BASE + 1


def _sc_body(t_hbm, c_hbm, r_hbm, p_hbm, out_hbm,
             t_v, c_v, r_v, x_v, y_v, z_v, a0_v, a1_v):
    f32 = jnp.float32
    i32 = jnp.int32
    wid = lax.axis_index("s") * _NC + lax.axis_index("c")
    nv = _BASE + jnp.where(wid < _REM, 1, 0)
    start = wid * _BASE + jnp.minimum(wid, _REM)

    pltpu.sync_copy(t_hbm, t_v)
    pltpu.sync_copy(c_hbm, c_v)
    pltpu.sync_copy(r_hbm, r_v)
    base = start * 16
    pltpu.sync_copy(p_hbm.at[pl.ds(base, _BASE * 16)],
                    x_v.at[pl.ds(0, _BASE * 16)])
    pltpu.sync_copy(p_hbm.at[pl.ds(_N + base, _BASE * 16)],
                    y_v.at[pl.ds(0, _BASE * 16)])
    pltpu.sync_copy(p_hbm.at[pl.ds(2 * _N + base, _BASE * 16)],
                    z_v.at[pl.ds(0, _BASE * 16)])

    @pl.when(wid < _REM)
    def _tail():
        e = base + _BASE * 16
        pltpu.sync_copy(p_hbm.at[pl.ds(e, 16)],
                        x_v.at[pl.ds(_BASE * 16, 16)])
        pltpu.sync_copy(p_hbm.at[pl.ds(_N + e, 16)],
                        y_v.at[pl.ds(_BASE * 16, 16)])
        pltpu.sync_copy(p_hbm.at[pl.ds(2 * _N + e, 16)],
                        z_v.at[pl.ds(_BASE * 16, 16)])

    gt = t_v[...]

    def coeffs(est_v):
        d = est_v[...] - gt
        return [jnp.full((16,), d[b * 4 + a], f32)
                for b in range(3) for a in range(4)]

    kc = coeffs(c_v)
    kr = coeffs(r_v)

    half = jnp.full((16,), 0.5, f32)
    c15 = jnp.full((16,), 1.5, f32)
    eps = jnp.full((16,), 1e-30, f32)
    magic = jnp.full((16,), 0x5F3759DF, i32)
    one_i = jnp.full((16,), 1, i32)

    def norm(k, x, y, z):
        e0 = k[0] * x + k[1] * y + k[2] * z + k[3]
        e1 = k[4] * x + k[5] * y + k[6] * z + k[7]
        e2 = k[8] * x + k[9] * y + k[10] * z + k[11]
        d2 = e0 * e0 + e1 * e1 + e2 * e2 + eps
        gi = magic - lax.shift_right_logical(plsc.bitcast(d2, i32), one_i)
        g = plsc.bitcast(gi, f32)
        for _ in range(3):
            g = g * (c15 - half * d2 * (g * g))
        return d2 * g

    def body(v, carry):
        ac, ar = carry
        off = v * 16
        x = x_v[pl.ds(off, 16)]
        y = y_v[pl.ds(off, 16)]
        z = z_v[pl.ds(off, 16)]
        w = jnp.full((16,), jnp.where(v < nv, f32(1.0), f32(0.0)), f32)
        ac = ac + norm(kc, x, y, z) * w
        ar = ar + norm(kr, x, y, z) * w
        return ac, ar

    zero = jnp.zeros((16,), f32)
    acc_c, acc_r = lax.fori_loop(0, _MAXV, body, (zero, zero))
    a0_v[...] = acc_c
    a1_v[...] = acc_r
    pltpu.sync_copy(a0_v, out_hbm.at[0, pl.ds(wid * 16, 16)])
    pltpu.sync_copy(a1_v, out_hbm.at[1, pl.ds(wid * 16, 16)])


def _acos(x):
    ax = jnp.abs(x)
    p = jnp.float32(-0.0012624911)
    for c in (0.0066700901, -0.0170881256, 0.0308918810, -0.0501743046,
              0.0889789874, -0.2145988016, 1.5707963050):
        p = p * ax + jnp.float32(c)
    r = jnp.sqrt(jnp.maximum(1.0 - ax, 0.0)) * p
    return jnp.where(x >= 0, r, jnp.float32(jnp.pi) - r)


def _fin_body(t_ref, c_ref, r_ref, v_ref, p_ref, out_ref):
    f32 = jnp.float32
    sums = jnp.sum(p_ref[...], axis=1)
    rmse_c = sums[0] * (1.0 / _N)
    rmse_r = sums[1] * (1.0 / _N)

    deg = f32(180.0 / jnp.pi)

    def scalars(est_ref):
        tr = f32(0.0)
        for b in range(3):
            for a in range(3):
                tr = tr + est_ref[b, a] * t_ref[b, a]
        x = jnp.clip(0.5 * (tr - 1.0), -1.0, 1.0)
        rre = _acos(x) * deg
        s2 = f32(0.0)
        for b in range(3):
            d = t_ref[b, 3] - est_ref[b, 3]
            s2 = s2 + d * d
        rte = jnp.sqrt(s2)
        recall = jnp.where((rre < 15.0) & (rte < 0.3), f32(1.0), f32(0.0))
        return rre, rte, recall

    rre_c, rte_c, recall_c = scalars(c_ref)
    rre_r, rte_r, recall_r = scalars(r_ref)

    out_ref[0] = rre_c
    out_ref[1] = rte_c
    out_ref[2] = rmse_c
    out_ref[3] = recall_c
    out_ref[4] = rre_r
    out_ref[5] = rte_r
    out_ref[6] = rmse_r
    out_ref[7] = recall_r
    out_ref[8] = v_ref[0]


@jax.jit
def kernel(transform_raw, coarse_trans, refined_trans, src_points, var_rt):
    transform = transform_raw[0]
    planes = src_points.T

    mesh = plsc.VectorSubcoreMesh(core_axis_name="c", subcore_axis_name="s")
    part = pl.kernel(
        _sc_body,
        mesh=mesh,
        compiler_params=pltpu.CompilerParams(needs_layout_passes=False),
        out_type=jax.ShapeDtypeStruct((2, _NW * 16), jnp.float32),
        scratch_types=[
            pltpu.VMEM((16,), jnp.float32),
            pltpu.VMEM((16,), jnp.float32),
            pltpu.VMEM((16,), jnp.float32),
            pltpu.VMEM((_MAXV * 16,), jnp.float32),
            pltpu.VMEM((_MAXV * 16,), jnp.float32),
            pltpu.VMEM((_MAXV * 16,), jnp.float32),
            pltpu.VMEM((16,), jnp.float32),
            pltpu.VMEM((16,), jnp.float32),
        ],
    )(transform.reshape(16), coarse_trans.reshape(16),
      refined_trans.reshape(16), planes.reshape(3 * _N))

    out = pl.pallas_call(
        _fin_body,
        in_specs=[pl.BlockSpec(memory_space=pltpu.SMEM)] * 4
        + [pl.BlockSpec(memory_space=pltpu.VMEM)],
        out_specs=pl.BlockSpec(memory_space=pltpu.SMEM),
        out_shape=jax.ShapeDtypeStruct((9,), jnp.float32),
    )(transform, coarse_trans, refined_trans, var_rt, part)
    return out


# speedup vs baseline: 7.5299x; 7.5299x over previous
import jax
import jax.numpy as jnp
from jax.experimental import pallas as pl
from jax.experimental.pallas import tpu as pltpu

_N = 100000


def _acos(x):
    ax = jnp.abs(x)
    p = jnp.float32(-0.0012624911)
    for c in (0.0066700901, -0.0170881256, 0.0308918810, -0.0501743046,
              0.0889789874, -0.2145988016, 1.5707963050):
        p = p * ax + jnp.float32(c)
    r = jnp.sqrt(jnp.maximum(1.0 - ax, 0.0)) * p
    return jnp.where(x >= 0, r, jnp.float32(jnp.pi) - r)


def _body(t_ref, c_ref, r_ref, v_ref, p_ref, out_ref):
    f32 = jnp.float32

    ji = jax.lax.broadcasted_iota(jnp.int32, (6, 3), 0)
    ai = jax.lax.broadcasted_iota(jnp.int32, (6, 3), 1)
    M = jnp.zeros((6, 3), f32)
    for h, est_ref in enumerate((c_ref, r_ref)):
        for b in range(3):
            for a in range(3):
                M = jnp.where((ji == 3 * h + b) & (ai == a),
                              est_ref[b, a] - t_ref[b, a], M)
    jb = jax.lax.broadcasted_iota(jnp.int32, (6, 1), 0)
    bias = jnp.zeros((6, 1), f32)
    for h, est_ref in enumerate((c_ref, r_ref)):
        for b in range(3):
            bias = jnp.where(jb == 3 * h + b, est_ref[b, 3] - t_ref[b, 3],
                             bias)
    qg = jax.lax.broadcasted_iota(jnp.int32, (2, 6), 0)
    jg = jax.lax.broadcasted_iota(jnp.int32, (2, 6), 1)
    S = ((jg // 3) == qg).astype(f32)

    P = p_ref[...]
    Y = jnp.dot(M, P, preferred_element_type=f32) + bias
    R = jnp.sqrt(jnp.dot(S, Y * Y, preferred_element_type=f32))
    sums = jnp.sum(R, axis=1)
    rmse_c = sums[0] * (1.0 / _N)
    rmse_r = sums[1] * (1.0 / _N)

    deg = f32(180.0 / jnp.pi)

    def scalars(est_ref):
        tr = f32(0.0)
        for b in range(3):
            for a in range(3):
                tr = tr + est_ref[b, a] * t_ref[b, a]
        x = jnp.clip(0.5 * (tr - 1.0), -1.0, 1.0)
        rre = _acos(x) * deg
        s2 = f32(0.0)
        for b in range(3):
            d = t_ref[b, 3] - est_ref[b, 3]
            s2 = s2 + d * d
        rte = jnp.sqrt(s2)
        recall = jnp.where((rre < 15.0) & (rte < 0.3), f32(1.0), f32(0.0))
        return rre, rte, recall

    rre_c, rte_c, recall_c = scalars(c_ref)
    rre_r, rte_r, recall_r = scalars(r_ref)

    out_ref[0] = rre_c
    out_ref[1] = rte_c
    out_ref[2] = rmse_c
    out_ref[3] = recall_c
    out_ref[4] = rre_r
    out_ref[5] = rte_r
    out_ref[6] = rmse_r
    out_ref[7] = recall_r
    out_ref[8] = v_ref[0]


@jax.jit
def kernel(transform_raw, coarse_trans, refined_trans, src_points, var_rt):
    transform = transform_raw[0]
    planes = src_points.T

    out = pl.pallas_call(
        _body,
        in_specs=[pl.BlockSpec(memory_space=pltpu.SMEM)] * 4
        + [pl.BlockSpec(memory_space=pltpu.VMEM)],
        out_specs=pl.BlockSpec(memory_space=pltpu.SMEM),
        out_shape=jax.ShapeDtypeStruct((9,), jnp.float32),
    )(transform, coarse_trans, refined_trans, var_rt, planes)
    return out
